# baseline (speedup 1.0000x reference)
import functools

import jax
import jax.numpy as jnp
from jax import lax
from jax.experimental import pallas as pl
from jax.experimental.pallas import tpu as pltpu
from jax.experimental.pallas import tpu_sc as plsc

N = 10000
E = 320000
D = 128

NC = 2
NS = 16
L = 16
NW = NC * NS
EPW = E // NW
CH = 40
NCHUNK = EPW // CH
NSET = 5
GD = NSET - 2
NP = 10240
RPT = NP // NS
DV = D // L


def _zero16():
    return jnp.zeros((L,), jnp.float32)


def _spmm_body(src_hbm, dst_hbm, ew_hbm, x_hbm, out_hbm, *scratch):
    sets = []
    for i in range(NSET):
        sets.append(scratch[i * 5:(i + 1) * 5])
    sems = scratch[NSET * 5:]
    for i in range(NSET):
        sets[i] = tuple(sets[i]) + (sems[3 * i], sems[3 * i + 1],
                                    sems[3 * i + 2])
    acc_sh = scratch[NSET * 5 + 3 * NSET]

    c = lax.axis_index("c")
    s = lax.axis_index("s")
    wid = s * NC + c
    ebase = wid * EPW
    row0 = s * RPT

    def _start_idx2(k, st):
        src_v, dstf_v, _, _, _, _, sem_i, _ = st
        off = ebase + k * CH
        pltpu.async_copy(src_hbm.at[pl.ds(off, CH)], src_v, sem_i)
        pltpu.async_copy(dst_hbm.at[pl.ds(off, CH)], dstf_v, sem_i)

    def _start_w(k, st):
        w_v = st[2]
        sem_i = st[6]
        off = ebase + k * CH
        pltpu.async_copy(ew_hbm.at[pl.ds(off, CH)], w_v, sem_i)

    def _start_idx(k, st):
        _start_idx2(k, st)
        _start_w(k, st)

    def _wait_idx(st):
        src_v, dstf_v, w_v, _, _, _, sem_i, _ = st
        pltpu.make_async_copy(src_hbm.at[pl.ds(0, CH)], src_v, sem_i).wait()
        pltpu.make_async_copy(dst_hbm.at[pl.ds(0, CH)], dstf_v, sem_i).wait()
        pltpu.make_async_copy(ew_hbm.at[pl.ds(0, CH)], w_v, sem_i).wait()

    def _start_gather(st):
        src_v, _, _, _, rows_v, sem_g, _, _ = st
        pltpu.async_copy(x_hbm.at[src_v], rows_v, sem_g)

    def _wait_gather(st):
        src_v, _, _, _, rows_v, sem_g, _, _ = st
        pltpu.make_async_copy(x_hbm.at[src_v], rows_v, sem_g).wait()

    def _start_scatter(st):
        _, _, _, dsts_v, rows_v, _, _, sem_s = st
        pltpu.async_copy(rows_v, acc_sh.at[dsts_v], sem_s, add=True)

    def _wait_scatter(st):
        _, _, _, dsts_v, rows_v, _, _, sem_s = st
        pltpu.make_async_copy(rows_v, acc_sh.at[dsts_v], sem_s).wait()

    def _scale(st):
        _, _, w_v, _, rows_v, _, _, _ = st
        for g in range(CH // L):
            wvec = w_v[pl.ds(g * L, L)]
            for t in range(L):
                e = g * L + t
                wscal = wvec[t]
                for j in range(DV):
                    sl = pl.ds(j * L, L)
                    rows_v[e, sl] = rows_v[e, sl] * wscal
        rem = CH - (CH // L) * L
        if rem:
            wvec = w_v[pl.ds(CH - L, L)]
            for t in range(L - rem, L):
                e = CH - L + t
                wscal = wvec[t]
                for j in range(DV):
                    sl = pl.ds(j * L, L)
                    rows_v[e, sl] = rows_v[e, sl] * wscal

    def _snapshot_dst(st):
        _, dstf_v, _, dsts_v, _, _, _, _ = st
        dsts_v[pl.ds(0, L)] = dstf_v[pl.ds(0, L)]
        dsts_v[pl.ds(L, L)] = dstf_v[pl.ds(L, L)]
        dsts_v[pl.ds(CH - L, L)] = dstf_v[pl.ds(CH - L, L)]

    def _phase(k, xset, cset):
        _wait_gather(xset)

        @pl.when(k >= 2)
        def _():
            _wait_scatter(cset)

        @pl.when(k + GD < NCHUNK)
        def _():
            _wait_idx(cset)
            _start_gather(cset)

        _scale(xset)
        _snapshot_dst(xset)
        _start_scatter(xset)

        @pl.when(k + NSET < NCHUNK)
        def _():
            _start_idx(k + NSET, xset)

    rows0 = sets[0][4]
    zsem = sets[0][5]
    for i in range(CH):
        for j in range(DV):
            rows0[i, pl.ds(j * L, L)] = _zero16()
    nz = RPT // CH
    for k in range(nz):
        pltpu.async_copy(rows0, acc_sh.at[pl.ds(row0 + k * CH, CH)], zsem)
    for k in range(nz):
        pltpu.make_async_copy(rows0, acc_sh.at[pl.ds(row0, CH)], zsem).wait()

    for i in range(NSET):
        _start_idx(i, sets[i])
    plsc.subcore_barrier()

    for i in range(GD):
        _wait_idx(sets[i])
        _start_gather(sets[i])

    def _round(kk, _):
        k0 = NSET * kk
        for j in range(NSET):
            _phase(k0 + j, sets[j], sets[(j + NSET - 2) % NSET])
        return 0
    nround = NCHUNK // NSET
    lax.fori_loop(0, nround, _round, 0)
    for k in range(nround * NSET, NCHUNK):
        j = k % NSET
        _phase(jnp.int32(k), sets[j], sets[(j + NSET - 2) % NSET])

    _wait_scatter(sets[(NCHUNK - 2) % NSET])
    _wait_scatter(sets[(NCHUNK - 1) % NSET])
    plsc.subcore_barrier()

    pltpu.sync_copy(acc_sh.at[pl.ds(row0, RPT)], out_hbm.at[c, pl.ds(row0, RPT)])


def _make_scratch():
    per_set = [
        pltpu.VMEM((CH,), jnp.int32),
        pltpu.VMEM((CH,), jnp.int32),
        pltpu.VMEM((CH,), jnp.float32),
        pltpu.VMEM((CH,), jnp.int32),
        pltpu.VMEM((CH, D), jnp.float32),
    ]
    scratch = []
    for _ in range(NSET):
        scratch.extend(per_set)
    for _ in range(3 * NSET):
        scratch.append(pltpu.SemaphoreType.DMA)
    scratch.append(pltpu.VMEM_SHARED((NP, D), jnp.float32))
    return scratch


@functools.cache
def _spmm_built():
    return pl.kernel(
        _spmm_body,
        out_type=jax.ShapeDtypeStruct((NC, NP, D), jnp.float32),
        mesh=plsc.VectorSubcoreMesh(core_axis_name="c", subcore_axis_name="s",
                                    num_cores=NC, num_subcores=NS),
        scratch_types=_make_scratch(),
    )


def _spmm(src, dst, ew, x):
    return _spmm_built()(src, dst, ew, x)


NB = 1000


def _linear_body(p_ref, w_ref, b_ref, o_ref):
    acc = p_ref[0] + p_ref[1]
    y = jnp.dot(acc, w_ref[...], preferred_element_type=jnp.float32)
    o_ref[...] = jnp.maximum(y + b_ref[...], 0.0)


def _linear(parts, W, b2d):
    return pl.pallas_call(
        _linear_body,
        grid=(N // NB,),
        in_specs=[
            pl.BlockSpec((NC, NB, D), lambda i: (0, i, 0)),
            pl.BlockSpec((D, D), lambda i: (0, 0)),
            pl.BlockSpec((1, D), lambda i: (0, 0)),
        ],
        out_specs=pl.BlockSpec((NB, D), lambda i: (i, 0)),
        out_shape=jax.ShapeDtypeStruct((N, D), jnp.float32),
    )(parts, W, b2d)


def kernel(node_features, edge_index, edge_weight, W1, b1, W2, b2):
    src = edge_index[0]
    dst = edge_index[1]
    x = node_features
    for W, b in ((W1, b1), (W2, b2)):
        parts = _spmm(src, dst, edge_weight, x)
        x = _linear(parts, W, b.reshape(1, D))
    return x


# speedup vs baseline: 1.1687x; 1.1687x over previous
import functools

import jax
import jax.numpy as jnp
from jax import lax
from jax.experimental import pallas as pl
from jax.experimental.pallas import tpu as pltpu
from jax.experimental.pallas import tpu_sc as plsc

N = 10000
E = 320000
D = 128

NC = 2
NS = 16
L = 16
NW = NC * NS
EPW = E // NW
CH = 40
NCHUNK = EPW // CH
NSET = 4
GD = NSET - 2
NP = 10240
RPT = NP // NS
DV = D // L


def _zero16():
    return jnp.zeros((L,), jnp.float32)


def _spmm_body(src_hbm, dst_hbm, ew_hbm, x_hbm, out_hbm, *scratch):
    sets = []
    for i in range(NSET):
        sets.append(scratch[i * 5:(i + 1) * 5])
    sems = scratch[NSET * 5:]
    for i in range(NSET):
        sets[i] = tuple(sets[i]) + (sems[3 * i], sems[3 * i + 1],
                                    sems[3 * i + 2])
    acc_sh = scratch[NSET * 5 + 3 * NSET]

    c = lax.axis_index("c")
    s = lax.axis_index("s")
    wid = s * NC + c
    ebase = wid * EPW
    row0 = s * RPT

    def _start_idx2(k, st):
        src_v, dstf_v, _, _, _, _, sem_i, _ = st
        off = ebase + k * CH
        pltpu.async_copy(src_hbm.at[pl.ds(off, CH)], src_v, sem_i)
        pltpu.async_copy(dst_hbm.at[pl.ds(off, CH)], dstf_v, sem_i)

    def _start_w(k, st):
        w_v = st[2]
        sem_i = st[6]
        off = ebase + k * CH
        pltpu.async_copy(ew_hbm.at[pl.ds(off, CH)], w_v, sem_i)

    def _start_idx(k, st):
        _start_idx2(k, st)
        _start_w(k, st)

    def _wait_idx(st):
        src_v, dstf_v, w_v, _, _, _, sem_i, _ = st
        pltpu.make_async_copy(src_hbm.at[pl.ds(0, CH)], src_v, sem_i).wait()
        pltpu.make_async_copy(dst_hbm.at[pl.ds(0, CH)], dstf_v, sem_i).wait()
        pltpu.make_async_copy(ew_hbm.at[pl.ds(0, CH)], w_v, sem_i).wait()

    def _start_gather(st):
        src_v, _, _, _, rows_v, sem_g, _, _ = st
        pltpu.async_copy(x_hbm.at[src_v], rows_v, sem_g)

    def _wait_gather(st):
        src_v, _, _, _, rows_v, sem_g, _, _ = st
        pltpu.make_async_copy(x_hbm.at[src_v], rows_v, sem_g).wait()

    def _start_scatter(st):
        _, _, _, dsts_v, rows_v, _, _, sem_s = st
        pltpu.async_copy(rows_v, acc_sh.at[dsts_v], sem_s, add=True)

    def _wait_scatter(st):
        _, _, _, dsts_v, rows_v, _, _, sem_s = st
        pltpu.make_async_copy(rows_v, acc_sh.at[dsts_v], sem_s).wait()

    def _scale(st):
        _, _, w_v, _, rows_v, _, _, _ = st
        for g in range(CH // L):
            wvec = w_v[pl.ds(g * L, L)]
            for t in range(L):
                e = g * L + t
                wscal = wvec[t]
                for j in range(DV):
                    sl = pl.ds(j * L, L)
                    rows_v[e, sl] = rows_v[e, sl] * wscal
        rem = CH - (CH // L) * L
        if rem:
            wvec = w_v[pl.ds(CH - L, L)]
            for t in range(L - rem, L):
                e = CH - L + t
                wscal = wvec[t]
                for j in range(DV):
                    sl = pl.ds(j * L, L)
                    rows_v[e, sl] = rows_v[e, sl] * wscal

    def _snapshot_dst(st):
        _, dstf_v, _, dsts_v, _, _, _, _ = st
        dsts_v[pl.ds(0, L)] = dstf_v[pl.ds(0, L)]
        dsts_v[pl.ds(L, L)] = dstf_v[pl.ds(L, L)]
        dsts_v[pl.ds(CH - L, L)] = dstf_v[pl.ds(CH - L, L)]

    def _phase(k, xset, cset):
        _wait_gather(xset)

        @pl.when(k >= 2)
        def _():
            _wait_scatter(cset)

        @pl.when(k + GD < NCHUNK)
        def _():
            _wait_idx(cset)
            _start_gather(cset)

        _scale(xset)
        _snapshot_dst(xset)
        _start_scatter(xset)

        @pl.when(k + NSET < NCHUNK)
        def _():
            _start_idx(k + NSET, xset)

    rows0 = sets[0][4]
    zsem = sets[0][5]
    for i in range(CH):
        for j in range(DV):
            rows0[i, pl.ds(j * L, L)] = _zero16()
    nz = RPT // CH
    for k in range(nz):
        pltpu.async_copy(rows0, acc_sh.at[pl.ds(row0 + k * CH, CH)], zsem)
    for k in range(nz):
        pltpu.make_async_copy(rows0, acc_sh.at[pl.ds(row0, CH)], zsem).wait()

    for i in range(NSET):
        _start_idx(i, sets[i])
    plsc.subcore_barrier()

    for i in range(GD):
        _wait_idx(sets[i])
        _start_gather(sets[i])

    def _round(kk, _):
        k0 = NSET * kk
        for j in range(NSET):
            _phase(k0 + j, sets[j], sets[(j + NSET - 2) % NSET])
        return 0
    nround = NCHUNK // NSET
    lax.fori_loop(0, nround, _round, 0)
    for k in range(nround * NSET, NCHUNK):
        j = k % NSET
        _phase(jnp.int32(k), sets[j], sets[(j + NSET - 2) % NSET])

    _wait_scatter(sets[(NCHUNK - 2) % NSET])
    _wait_scatter(sets[(NCHUNK - 1) % NSET])
    plsc.subcore_barrier()

    pltpu.sync_copy(acc_sh.at[pl.ds(row0, RPT)], out_hbm.at[c, pl.ds(row0, RPT)])


def _make_scratch():
    per_set = [
        pltpu.VMEM((CH,), jnp.int32),
        pltpu.VMEM((CH,), jnp.int32),
        pltpu.VMEM((CH,), jnp.float32),
        pltpu.VMEM((CH,), jnp.int32),
        pltpu.VMEM((CH, D), jnp.float32),
    ]
    scratch = []
    for _ in range(NSET):
        scratch.extend(per_set)
    for _ in range(3 * NSET):
        scratch.append(pltpu.SemaphoreType.DMA)
    scratch.append(pltpu.VMEM_SHARED((NP, D), jnp.float32))
    return scratch


@functools.cache
def _spmm_built():
    return pl.kernel(
        _spmm_body,
        out_type=jax.ShapeDtypeStruct((NC, NP, D), jnp.float32),
        mesh=plsc.VectorSubcoreMesh(core_axis_name="c", subcore_axis_name="s",
                                    num_cores=NC, num_subcores=NS),
        scratch_types=_make_scratch(),
    )


def _spmm(src, dst, ew, x):
    return _spmm_built()(src, dst, ew, x)


NB = 1000


def _linear_body(p_ref, w_ref, b_ref, o_ref):
    acc = p_ref[0] + p_ref[1]
    y = jnp.dot(acc, w_ref[...], preferred_element_type=jnp.float32)
    o_ref[...] = jnp.maximum(y + b_ref[...], 0.0)


def _linear(parts, W, b2d):
    return pl.pallas_call(
        _linear_body,
        grid=(N // NB,),
        in_specs=[
            pl.BlockSpec((NC, NB, D), lambda i: (0, i, 0)),
            pl.BlockSpec((D, D), lambda i: (0, 0)),
            pl.BlockSpec((1, D), lambda i: (0, 0)),
        ],
        out_specs=pl.BlockSpec((NB, D), lambda i: (i, 0)),
        out_shape=jax.ShapeDtypeStruct((N, D), jnp.float32),
    )(parts, W, b2d)


def kernel(node_features, edge_index, edge_weight, W1, b1, W2, b2):
    src = edge_index[0]
    dst = edge_index[1]
    x = node_features
    for W, b in ((W1, b1), (W2, b2)):
        parts = _spmm(src, dst, edge_weight, x)
        x = _linear(parts, W, b.reshape(1, D))
    return x
